# baseline (speedup 1.0000x reference)
import jax
import jax.numpy as jnp
from jax import lax
from jax.experimental import pallas as pl
from jax.experimental.pallas import tpu as pltpu
from jax.experimental.pallas import tpu_sc as plsc

N_ROWS = 10000
N_ACC = 10240
D = 256
DH = 128
E = 160000
NT = 16
C = 128
CPT = 80
E_PAD = NT * CPT * C
RPT = N_ACC // NT


def _sc_lightgcn(bot, usr, top, ujr, ujc, ujv, ijr, ijc, ijv):
    mesh = plsc.VectorSubcoreMesh(core_axis_name="c", subcore_axis_name="s")
    f32 = jnp.float32
    bf16 = jnp.bfloat16

    def body(bot_hbm, usr_hbm, top_hbm,
             ujr_hbm, ujc_hbm, ujv_hbm, ijr_hbm, ijc_hbm, ijv_hbm,
             out_u_hbm, out_t_hbm, out_p_hbm,
             acc, rows_v, cols_v, vals_v, gb0, gb1, gb2, gb3,
             sg0, sg1, sg2, sg3, ss0, ss1, ss2, ss3):
        cid = lax.axis_index("c")
        tid = lax.axis_index("s")
        half_off = cid * N_ROWS
        gb = (gb0, gb1, gb2, gb3)
        sg = (sg0, sg1, sg2, sg3)
        ss = (ss0, ss1, ss2, ss3)

        z32 = jnp.zeros((32,), bf16)

        def zero_acc():
            def zfill(r, carry):
                for c32 in range(DH // 32):
                    gb0[r, pl.ds(c32 * 32, 32)] = z32
                return carry
            lax.fori_loop(0, C, zfill, 0)
            for k in range(RPT // C):
                pltpu.sync_copy(gb0, acc.at[pl.ds(tid * RPT + k * C, C)])

        def accumulate(rows_hbm, cols_hbm, vals_hbm, table_hbm):
            base = tid * CPT

            def start_g(g, b):
                pltpu.async_copy(table_hbm.at[cols_v.at[g]], gb[b], sg[b])

            def wait_g(b):
                pltpu.make_async_copy(table_hbm.at[cols_v.at[0]], gb[b],
                                      sg[b]).wait()

            def start_s(g, b):
                pltpu.async_copy(gb[b], acc.at[rows_v.at[g]], ss[b], add=True)

            def wait_s(b):
                pltpu.make_async_copy(gb[b], acc.at[rows_v.at[0]],
                                      ss[b]).wait()

            dnums = lax.GatherDimensionNumbers(
                offset_dims=(), collapsed_slice_dims=(0,),
                start_index_map=(0,))

            def scale(g, b):
                buf = gb[b]

                def egroup(q, c2):
                    vv = vals_v[g, pl.ds(q * 16, 16)]

                    def lanes(lane, c3):
                        bidx = jnp.full((16,), lane, jnp.int32)
                        v16 = lax.gather(
                            vv, bidx[:, None], dnums, (1,),
                            mode=lax.GatherScatterMode.PROMISE_IN_BOUNDS)
                        v32 = plsc.pack(v16, v16,
                                        format=plsc.PackFormat.INTERLEAVED)
                        e = q * 16 + lane
                        for d32 in range(DH // 32):
                            sl = pl.ds(d32 * 32, 32)
                            buf[e, sl] = buf[e, sl] * v32
                        return c3
                    return lax.fori_loop(0, 16, lanes, c2)
                lax.fori_loop(0, C // 16, egroup, 0)

            pltpu.sync_copy(rows_hbm.at[pl.ds(base, CPT)], rows_v)
            pltpu.sync_copy(cols_hbm.at[pl.ds(base, CPT)], cols_v)
            pltpu.sync_copy(vals_hbm.at[pl.ds(base, CPT)], vals_v)

            off16 = jnp.full((16,), half_off, jnp.int32)

            def fix(i, carry):
                r = i // (C // 16)
                c = (i % (C // 16)) * 16
                cols_v[r, pl.ds(c, 16)] = cols_v[r, pl.ds(c, 16)] + off16
                return carry
            lax.fori_loop(0, CPT * (C // 16), fix, 0)

            start_g(0, 0)
            start_g(1, 1)

            def rnd(r, carry):
                for j in range(4):
                    g = 4 * r + j
                    b = j
                    wait_g(b)
                    scale(g, b)
                    start_s(g, b)

                    @pl.when(g >= 2)
                    def _():
                        wait_s((b + 2) % 4)

                    @pl.when(g + 2 < CPT)
                    def _():
                        start_g(g + 2, (b + 2) % 4)
                return carry
            lax.fori_loop(0, CPT // 4, rnd, 0)
            wait_s((CPT - 2) % 4)
            wait_s((CPT - 1) % 4)

        def writeback(out_hbm):
            r0 = tid * RPT
            pltpu.sync_copy(acc.at[pl.ds(r0, RPT)],
                            out_hbm.at[pl.ds(cid * N_ACC + r0, RPT)])

        zero_acc()
        plsc.subcore_barrier()
        accumulate(ujr_hbm, ujc_hbm, ujv_hbm, bot_hbm)
        plsc.subcore_barrier()
        writeback(out_u_hbm)

        zero_acc()
        plsc.subcore_barrier()
        accumulate(ijr_hbm, ijc_hbm, ijv_hbm, bot_hbm)
        plsc.subcore_barrier()
        writeback(out_t_hbm)

        zero_acc()
        plsc.subcore_barrier()
        accumulate(ujc_hbm, ujr_hbm, ujv_hbm, usr_hbm)
        accumulate(ijc_hbm, ijr_hbm, ijv_hbm, top_hbm)
        plsc.subcore_barrier()
        writeback(out_p_hbm)

    out_sds = jax.ShapeDtypeStruct((2 * N_ACC, DH), bf16)
    run = pl.kernel(
        body,
        out_type=(out_sds, out_sds, out_sds),
        mesh=mesh,
        compiler_params=pltpu.CompilerParams(use_tc_tiling_on_sc=False,
                                             needs_layout_passes=False),
        scratch_types=(
            pltpu.VMEM_SHARED((N_ACC, DH), bf16),
            pltpu.VMEM((CPT, C), jnp.int32),
            pltpu.VMEM((CPT, C), jnp.int32),
            pltpu.VMEM((CPT, C), f32),
            pltpu.VMEM((C, DH), bf16),
            pltpu.VMEM((C, DH), bf16),
            pltpu.VMEM((C, DH), bf16),
            pltpu.VMEM((C, DH), bf16),
            pltpu.SemaphoreType.DMA,
            pltpu.SemaphoreType.DMA,
            pltpu.SemaphoreType.DMA,
            pltpu.SemaphoreType.DMA,
            pltpu.SemaphoreType.DMA,
            pltpu.SemaphoreType.DMA,
            pltpu.SemaphoreType.DMA,
            pltpu.SemaphoreType.DMA,
        ),
    )
    return run(bot, usr, top, ujr, ujc, ujv, ijr, ijc, ijv)


def kernel(adj_UJ_indices, adj_UJ_values, adj_IJ_indices, adj_IJ_values,
           top_embs, pos_bottoms_embs, all_users_embs):
    i32 = jnp.int32

    def pad_idx(x):
        return jnp.pad(x.astype(i32), (0, E_PAD - E)).reshape(E_PAD // C, C)

    def pad_val(x):
        return jnp.pad(x, (0, E_PAD - E)).reshape(E_PAD // C, C)

    ujr = pad_idx(adj_UJ_indices[0])
    ujc = pad_idx(adj_UJ_indices[1])
    ijr = pad_idx(adj_IJ_indices[0])
    ijc = pad_idx(adj_IJ_indices[1])
    ujv = pad_val(adj_UJ_values)
    ijv = pad_val(adj_IJ_values)

    def stack_halves(x):
        return jnp.concatenate([x[:, :DH], x[:, DH:]],
                               axis=0).astype(jnp.bfloat16)

    bot = stack_halves(pos_bottoms_embs)
    usr = stack_halves(all_users_embs)
    top = stack_halves(top_embs)

    out_u, out_t, out_p = _sc_lightgcn(bot, usr, top, ujr, ujc, ujv,
                                       ijr, ijc, ijv)

    def unstack(o):
        return jnp.concatenate([o[:N_ROWS], o[N_ACC:N_ACC + N_ROWS]],
                               axis=1).astype(jnp.float32)

    return (unstack(out_u), unstack(out_t), unstack(out_p))


# speedup vs baseline: 1.0381x; 1.0381x over previous
import jax
import jax.numpy as jnp
from jax import lax
from jax.experimental import pallas as pl
from jax.experimental.pallas import tpu as pltpu
from jax.experimental.pallas import tpu_sc as plsc

N_ROWS = 10000
N_ACC = 10240
D = 256
DH = 128
E = 160000
NT = 16
C = 128
CPT = 80
E_PAD = NT * CPT * C
RPT = N_ACC // NT


def _sc_lightgcn(bot, usr, top, ujr, ujc, ujv, ijr, ijc, ijv):
    mesh = plsc.VectorSubcoreMesh(core_axis_name="c", subcore_axis_name="s")
    f32 = jnp.float32
    bf16 = jnp.bfloat16

    def body(bot_hbm, usr_hbm, top_hbm,
             ujr_hbm, ujc_hbm, ujv_hbm, ijr_hbm, ijc_hbm, ijv_hbm,
             out_u_hbm, out_t_hbm, out_p_hbm,
             acc, rows_v, cols_v, vals_v, gb0, gb1, gb2, gb3, gb4,
             sg0, sg1, sg2, sg3, sg4, ss0, ss1, ss2, ss3, ss4):
        cid = lax.axis_index("c")
        tid = lax.axis_index("s")
        half_off = cid * N_ROWS
        gb = (gb0, gb1, gb2, gb3, gb4)
        sg = (sg0, sg1, sg2, sg3, sg4)
        ss = (ss0, ss1, ss2, ss3, ss4)

        z32 = jnp.zeros((32,), bf16)

        def zero_acc():
            def zfill(r, carry):
                for c32 in range(DH // 32):
                    gb0[r, pl.ds(c32 * 32, 32)] = z32
                return carry
            lax.fori_loop(0, C, zfill, 0)
            for k in range(RPT // C):
                pltpu.sync_copy(gb0, acc.at[pl.ds(tid * RPT + k * C, C)])

        def accumulate(rows_hbm, cols_hbm, vals_hbm, table_hbm):
            base = tid * CPT

            def start_g(g, b):
                pltpu.async_copy(table_hbm.at[cols_v.at[g]], gb[b], sg[b])

            def wait_g(b):
                pltpu.make_async_copy(table_hbm.at[cols_v.at[0]], gb[b],
                                      sg[b]).wait()

            def start_s(g, b):
                pltpu.async_copy(gb[b], acc.at[rows_v.at[g]], ss[b], add=True)

            def wait_s(b):
                pltpu.make_async_copy(gb[b], acc.at[rows_v.at[0]],
                                      ss[b]).wait()

            dnums = lax.GatherDimensionNumbers(
                offset_dims=(), collapsed_slice_dims=(0,),
                start_index_map=(0,))

            def scale(g, b):
                buf = gb[b]

                def egroup(q, c2):
                    vv = vals_v[g, pl.ds(q * 16, 16)]

                    def lanes(lane, c3):
                        bidx = jnp.full((16,), lane, jnp.int32)
                        v16 = lax.gather(
                            vv, bidx[:, None], dnums, (1,),
                            mode=lax.GatherScatterMode.PROMISE_IN_BOUNDS)
                        v32 = plsc.pack(v16, v16,
                                        format=plsc.PackFormat.INTERLEAVED)
                        e = q * 16 + lane
                        for d32 in range(DH // 32):
                            sl = pl.ds(d32 * 32, 32)
                            buf[e, sl] = buf[e, sl] * v32
                        return c3
                    return lax.fori_loop(0, 16, lanes, c2)
                lax.fori_loop(0, C // 16, egroup, 0)

            pltpu.sync_copy(rows_hbm.at[pl.ds(base, CPT)], rows_v)
            pltpu.sync_copy(cols_hbm.at[pl.ds(base, CPT)], cols_v)
            pltpu.sync_copy(vals_hbm.at[pl.ds(base, CPT)], vals_v)

            off16 = jnp.full((16,), half_off, jnp.int32)

            def fix(i, carry):
                r = i // (C // 16)
                c = (i % (C // 16)) * 16
                cols_v[r, pl.ds(c, 16)] = cols_v[r, pl.ds(c, 16)] + off16
                return carry
            lax.fori_loop(0, CPT * (C // 16), fix, 0)

            start_g(0, 0)
            start_g(1, 1)
            start_g(2, 2)

            def rnd(r, carry):
                for j in range(5):
                    g = 5 * r + j
                    b = j
                    wait_g(b)
                    scale(g, b)
                    start_s(g, b)

                    @pl.when(g >= 2)
                    def _():
                        wait_s((b + 3) % 5)

                    @pl.when(g + 3 < CPT)
                    def _():
                        start_g(g + 3, (b + 3) % 5)
                return carry
            lax.fori_loop(0, CPT // 5, rnd, 0)
            wait_s((CPT - 2) % 5)
            wait_s((CPT - 1) % 5)

        def writeback(out_hbm):
            r0 = tid * RPT
            pltpu.sync_copy(acc.at[pl.ds(r0, RPT)],
                            out_hbm.at[pl.ds(cid * N_ACC + r0, RPT)])

        zero_acc()
        plsc.subcore_barrier()
        accumulate(ujr_hbm, ujc_hbm, ujv_hbm, bot_hbm)
        plsc.subcore_barrier()
        writeback(out_u_hbm)

        zero_acc()
        plsc.subcore_barrier()
        accumulate(ijr_hbm, ijc_hbm, ijv_hbm, bot_hbm)
        plsc.subcore_barrier()
        writeback(out_t_hbm)

        zero_acc()
        plsc.subcore_barrier()
        accumulate(ujc_hbm, ujr_hbm, ujv_hbm, usr_hbm)
        accumulate(ijc_hbm, ijr_hbm, ijv_hbm, top_hbm)
        plsc.subcore_barrier()
        writeback(out_p_hbm)

    out_sds = jax.ShapeDtypeStruct((2 * N_ACC, DH), bf16)
    run = pl.kernel(
        body,
        out_type=(out_sds, out_sds, out_sds),
        mesh=mesh,
        compiler_params=pltpu.CompilerParams(use_tc_tiling_on_sc=False,
                                             needs_layout_passes=False),
        scratch_types=(
            pltpu.VMEM_SHARED((N_ACC, DH), bf16),
            pltpu.VMEM((CPT, C), jnp.int32),
            pltpu.VMEM((CPT, C), jnp.int32),
            pltpu.VMEM((CPT, C), f32),
            pltpu.VMEM((C, DH), bf16),
            pltpu.VMEM((C, DH), bf16),
            pltpu.VMEM((C, DH), bf16),
            pltpu.VMEM((C, DH), bf16),
            pltpu.VMEM((C, DH), bf16),
            pltpu.SemaphoreType.DMA,
            pltpu.SemaphoreType.DMA,
            pltpu.SemaphoreType.DMA,
            pltpu.SemaphoreType.DMA,
            pltpu.SemaphoreType.DMA,
            pltpu.SemaphoreType.DMA,
            pltpu.SemaphoreType.DMA,
            pltpu.SemaphoreType.DMA,
            pltpu.SemaphoreType.DMA,
            pltpu.SemaphoreType.DMA,
        ),
    )
    return run(bot, usr, top, ujr, ujc, ujv, ijr, ijc, ijv)


def kernel(adj_UJ_indices, adj_UJ_values, adj_IJ_indices, adj_IJ_values,
           top_embs, pos_bottoms_embs, all_users_embs):
    i32 = jnp.int32

    def pad_idx(x):
        return jnp.pad(x.astype(i32), (0, E_PAD - E)).reshape(E_PAD // C, C)

    def pad_val(x):
        return jnp.pad(x, (0, E_PAD - E)).reshape(E_PAD // C, C)

    ujr = pad_idx(adj_UJ_indices[0])
    ujc = pad_idx(adj_UJ_indices[1])
    ijr = pad_idx(adj_IJ_indices[0])
    ijc = pad_idx(adj_IJ_indices[1])
    ujv = pad_val(adj_UJ_values)
    ijv = pad_val(adj_IJ_values)

    def stack_halves(x):
        return jnp.concatenate([x[:, :DH], x[:, DH:]],
                               axis=0).astype(jnp.bfloat16)

    bot = stack_halves(pos_bottoms_embs)
    usr = stack_halves(all_users_embs)
    top = stack_halves(top_embs)

    out_u, out_t, out_p = _sc_lightgcn(bot, usr, top, ujr, ujc, ujv,
                                       ijr, ijc, ijv)

    def unstack(o):
        return jnp.concatenate([o[:N_ROWS], o[N_ACC:N_ACC + N_ROWS]],
                               axis=1).astype(jnp.float32)

    return (unstack(out_u), unstack(out_t), unstack(out_p))


# speedup vs baseline: 1.0396x; 1.0015x over previous
import jax
import jax.numpy as jnp
from jax import lax
from jax.experimental import pallas as pl
from jax.experimental.pallas import tpu as pltpu
from jax.experimental.pallas import tpu_sc as plsc

N_ROWS = 10000
N_ACC = 10240
D = 256
DH = 128
E = 160000
NT = 16
C = 128
CPT = 80
E_PAD = NT * CPT * C
RPT = N_ACC // NT


def _sc_lightgcn(bot, usr, top, ujr, ujc, ujv, ijr, ijc, ijv):
    mesh = plsc.VectorSubcoreMesh(core_axis_name="c", subcore_axis_name="s")
    f32 = jnp.float32
    bf16 = jnp.bfloat16

    def body(bot_hbm, usr_hbm, top_hbm,
             ujr_hbm, ujc_hbm, ujv_hbm, ijr_hbm, ijc_hbm, ijv_hbm,
             out_u_hbm, out_t_hbm, out_p_hbm,
             acc, rows_v, cols_v, vals_v, gb0, gb1, gb2, gb3, gb4, zbuf,
             sg0, sg1, sg2, sg3, sg4, ss0, ss1, ss2, ss3, ss4, sz):
        cid = lax.axis_index("c")
        tid = lax.axis_index("s")
        half_off = cid * N_ROWS
        gb = (gb0, gb1, gb2, gb3, gb4)
        sg = (sg0, sg1, sg2, sg3, sg4)
        ss = (ss0, ss1, ss2, ss3, ss4)

        z32 = jnp.zeros((32,), bf16)

        def zfill(r, carry):
            for c32 in range(DH // 32):
                zbuf[r, pl.ds(c32 * 32, 32)] = z32
            return carry
        lax.fori_loop(0, C, zfill, 0)

        def zero_acc():
            for k in range(RPT // C):
                pltpu.async_copy(zbuf, acc.at[pl.ds(tid * RPT + k * C, C)],
                                 sz)
            for k in range(RPT // C):
                pltpu.make_async_copy(zbuf,
                                      acc.at[pl.ds(tid * RPT + k * C, C)],
                                      sz).wait()

        def accumulate(rows_hbm, cols_hbm, vals_hbm, table_hbm):
            base = tid * CPT

            def start_g(g, b):
                pltpu.async_copy(table_hbm.at[cols_v.at[g]], gb[b], sg[b])

            def wait_g(b):
                pltpu.make_async_copy(table_hbm.at[cols_v.at[0]], gb[b],
                                      sg[b]).wait()

            def start_s(g, b):
                pltpu.async_copy(gb[b], acc.at[rows_v.at[g]], ss[b], add=True)

            def wait_s(b):
                pltpu.make_async_copy(gb[b], acc.at[rows_v.at[0]],
                                      ss[b]).wait()

            dnums = lax.GatherDimensionNumbers(
                offset_dims=(), collapsed_slice_dims=(0,),
                start_index_map=(0,))

            def scale(g, b):
                buf = gb[b]

                def egroup(q, c2):
                    vv = vals_v[g, pl.ds(q * 16, 16)]

                    def lanes(lane, c3):
                        bidx = jnp.full((16,), lane, jnp.int32)
                        v16 = lax.gather(
                            vv, bidx[:, None], dnums, (1,),
                            mode=lax.GatherScatterMode.PROMISE_IN_BOUNDS)
                        v32 = plsc.pack(v16, v16,
                                        format=plsc.PackFormat.INTERLEAVED)
                        e = q * 16 + lane
                        for d32 in range(DH // 32):
                            sl = pl.ds(d32 * 32, 32)
                            buf[e, sl] = buf[e, sl] * v32
                        return c3
                    return lax.fori_loop(0, 16, lanes, c2)
                lax.fori_loop(0, C // 16, egroup, 0)

            pltpu.sync_copy(rows_hbm.at[pl.ds(base, CPT)], rows_v)
            pltpu.sync_copy(cols_hbm.at[pl.ds(base, CPT)], cols_v)
            pltpu.sync_copy(vals_hbm.at[pl.ds(base, CPT)], vals_v)

            off16 = jnp.full((16,), half_off, jnp.int32)

            def fix(i, carry):
                r = i // (C // 16)
                c = (i % (C // 16)) * 16
                cols_v[r, pl.ds(c, 16)] = cols_v[r, pl.ds(c, 16)] + off16
                return carry
            lax.fori_loop(0, CPT * (C // 16), fix, 0)

            start_g(0, 0)
            start_g(1, 1)
            start_g(2, 2)

            def rnd(r, carry):
                for j in range(5):
                    g = 5 * r + j
                    b = j
                    wait_g(b)
                    scale(g, b)
                    start_s(g, b)

                    @pl.when(g >= 2)
                    def _():
                        wait_s((b + 3) % 5)

                    @pl.when(g + 3 < CPT)
                    def _():
                        start_g(g + 3, (b + 3) % 5)
                return carry
            lax.fori_loop(0, CPT // 5, rnd, 0)
            wait_s((CPT - 2) % 5)
            wait_s((CPT - 1) % 5)

        def writeback(out_hbm):
            r0 = tid * RPT
            pltpu.sync_copy(acc.at[pl.ds(r0, RPT)],
                            out_hbm.at[pl.ds(cid * N_ACC + r0, RPT)])

        zero_acc()
        plsc.subcore_barrier()
        accumulate(ujr_hbm, ujc_hbm, ujv_hbm, bot_hbm)
        plsc.subcore_barrier()
        writeback(out_u_hbm)

        zero_acc()
        plsc.subcore_barrier()
        accumulate(ijr_hbm, ijc_hbm, ijv_hbm, bot_hbm)
        plsc.subcore_barrier()
        writeback(out_t_hbm)

        zero_acc()
        plsc.subcore_barrier()
        accumulate(ujc_hbm, ujr_hbm, ujv_hbm, usr_hbm)
        accumulate(ijc_hbm, ijr_hbm, ijv_hbm, top_hbm)
        plsc.subcore_barrier()
        writeback(out_p_hbm)

    out_sds = jax.ShapeDtypeStruct((2 * N_ACC, DH), bf16)
    run = pl.kernel(
        body,
        out_type=(out_sds, out_sds, out_sds),
        mesh=mesh,
        compiler_params=pltpu.CompilerParams(use_tc_tiling_on_sc=False,
                                             needs_layout_passes=False),
        scratch_types=(
            pltpu.VMEM_SHARED((N_ACC, DH), bf16),
            pltpu.VMEM((CPT, C), jnp.int32),
            pltpu.VMEM((CPT, C), jnp.int32),
            pltpu.VMEM((CPT, C), f32),
            pltpu.VMEM((C, DH), bf16),
            pltpu.VMEM((C, DH), bf16),
            pltpu.VMEM((C, DH), bf16),
            pltpu.VMEM((C, DH), bf16),
            pltpu.VMEM((C, DH), bf16),
            pltpu.VMEM((C, DH), bf16),
            pltpu.SemaphoreType.DMA,
            pltpu.SemaphoreType.DMA,
            pltpu.SemaphoreType.DMA,
            pltpu.SemaphoreType.DMA,
            pltpu.SemaphoreType.DMA,
            pltpu.SemaphoreType.DMA,
            pltpu.SemaphoreType.DMA,
            pltpu.SemaphoreType.DMA,
            pltpu.SemaphoreType.DMA,
            pltpu.SemaphoreType.DMA,
            pltpu.SemaphoreType.DMA,
        ),
    )
    return run(bot, usr, top, ujr, ujc, ujv, ijr, ijc, ijv)


def kernel(adj_UJ_indices, adj_UJ_values, adj_IJ_indices, adj_IJ_values,
           top_embs, pos_bottoms_embs, all_users_embs):
    i32 = jnp.int32

    def pad_idx(x):
        return jnp.pad(x.astype(i32), (0, E_PAD - E)).reshape(E_PAD // C, C)

    def pad_val(x):
        return jnp.pad(x, (0, E_PAD - E)).reshape(E_PAD // C, C)

    ujr = pad_idx(adj_UJ_indices[0])
    ujc = pad_idx(adj_UJ_indices[1])
    ijr = pad_idx(adj_IJ_indices[0])
    ijc = pad_idx(adj_IJ_indices[1])
    ujv = pad_val(adj_UJ_values)
    ijv = pad_val(adj_IJ_values)

    def stack_halves(x):
        return jnp.concatenate([x[:, :DH], x[:, DH:]],
                               axis=0).astype(jnp.bfloat16)

    bot = stack_halves(pos_bottoms_embs)
    usr = stack_halves(all_users_embs)
    top = stack_halves(top_embs)

    out_u, out_t, out_p = _sc_lightgcn(bot, usr, top, ujr, ujc, ujv,
                                       ijr, ijc, ijv)

    def unstack(o):
        return jnp.concatenate([o[:N_ROWS], o[N_ACC:N_ACC + N_ROWS]],
                               axis=1).astype(jnp.float32)

    return (unstack(out_u), unstack(out_t), unstack(out_p))


# speedup vs baseline: 1.0437x; 1.0039x over previous
import jax
import jax.numpy as jnp
from jax import lax
from jax.experimental import pallas as pl
from jax.experimental.pallas import tpu as pltpu
from jax.experimental.pallas import tpu_sc as plsc

N_ROWS = 10000
N_ACC = 10240
D = 256
DH = 128
E = 160000
NT = 16
C = 128
CPT = 80
E_PAD = NT * CPT * C
RPT = N_ACC // NT


def _sc_lightgcn(bot, usr, top, ujr, ujc, ujv, ijr, ijc, ijv):
    mesh = plsc.VectorSubcoreMesh(core_axis_name="c", subcore_axis_name="s")
    f32 = jnp.float32
    bf16 = jnp.bfloat16

    def body(bot_hbm, usr_hbm, top_hbm,
             ujr_hbm, ujc_hbm, ujv_hbm, ijr_hbm, ijc_hbm, ijv_hbm,
             out_u_hbm, out_t_hbm, out_p_hbm,
             acc, rows_v, cols_v, vals_v, gb0, gb1, gb2, gb3, gb4, zbuf,
             sg0, sg1, sg2, sg3, sg4, ss0, ss1, ss2, ss3, ss4, sz):
        cid = lax.axis_index("c")
        tid = lax.axis_index("s")
        half_off = cid * N_ROWS
        gb = (gb0, gb1, gb2, gb3, gb4)
        sg = (sg0, sg1, sg2, sg3, sg4)
        ss = (ss0, ss1, ss2, ss3, ss4)

        z32 = jnp.zeros((32,), bf16)

        def zfill(r, carry):
            for c32 in range(DH // 32):
                zbuf[r, pl.ds(c32 * 32, 32)] = z32
            return carry
        lax.fori_loop(0, C, zfill, 0)

        def zero_fire():
            for k in range(RPT // C):
                pltpu.async_copy(zbuf, acc.at[pl.ds(tid * RPT + k * C, C)],
                                 sz)

        def zero_drain():
            for k in range(RPT // C):
                pltpu.make_async_copy(zbuf,
                                      acc.at[pl.ds(tid * RPT + k * C, C)],
                                      sz).wait()

        def load_idx(rows_hbm, cols_hbm, vals_hbm):
            base = tid * CPT
            pltpu.sync_copy(rows_hbm.at[pl.ds(base, CPT)], rows_v)
            pltpu.sync_copy(cols_hbm.at[pl.ds(base, CPT)], cols_v)
            pltpu.sync_copy(vals_hbm.at[pl.ds(base, CPT)], vals_v)

            off16 = jnp.full((16,), half_off, jnp.int32)

            def fix(i, carry):
                r = i // (C // 16)
                c = (i % (C // 16)) * 16
                cols_v[r, pl.ds(c, 16)] = cols_v[r, pl.ds(c, 16)] + off16
                return carry
            lax.fori_loop(0, CPT * (C // 16), fix, 0)

        def accumulate(table_hbm):

            def start_g(g, b):
                pltpu.async_copy(table_hbm.at[cols_v.at[g]], gb[b], sg[b])

            def wait_g(b):
                pltpu.make_async_copy(table_hbm.at[cols_v.at[0]], gb[b],
                                      sg[b]).wait()

            def start_s(g, b):
                pltpu.async_copy(gb[b], acc.at[rows_v.at[g]], ss[b], add=True)

            def wait_s(b):
                pltpu.make_async_copy(gb[b], acc.at[rows_v.at[0]],
                                      ss[b]).wait()

            dnums = lax.GatherDimensionNumbers(
                offset_dims=(), collapsed_slice_dims=(0,),
                start_index_map=(0,))

            def scale(g, b):
                buf = gb[b]

                def egroup(q, c2):
                    vv = vals_v[g, pl.ds(q * 16, 16)]

                    def lanes(lane, c3):
                        bidx = jnp.full((16,), lane, jnp.int32)
                        v16 = lax.gather(
                            vv, bidx[:, None], dnums, (1,),
                            mode=lax.GatherScatterMode.PROMISE_IN_BOUNDS)
                        v32 = plsc.pack(v16, v16,
                                        format=plsc.PackFormat.INTERLEAVED)
                        e = q * 16 + lane
                        for d32 in range(DH // 32):
                            sl = pl.ds(d32 * 32, 32)
                            buf[e, sl] = buf[e, sl] * v32
                        return c3
                    return lax.fori_loop(0, 16, lanes, c2)
                lax.fori_loop(0, C // 16, egroup, 0)

            start_g(0, 0)
            start_g(1, 1)
            start_g(2, 2)

            def rnd(r, carry):
                for j in range(5):
                    g = 5 * r + j
                    b = j
                    wait_g(b)
                    scale(g, b)
                    start_s(g, b)

                    @pl.when(g >= 2)
                    def _():
                        wait_s((b + 3) % 5)

                    @pl.when(g + 3 < CPT)
                    def _():
                        start_g(g + 3, (b + 3) % 5)
                return carry
            lax.fori_loop(0, CPT // 5, rnd, 0)
            wait_s((CPT - 2) % 5)
            wait_s((CPT - 1) % 5)

        def writeback(out_hbm):
            r0 = tid * RPT
            pltpu.sync_copy(acc.at[pl.ds(r0, RPT)],
                            out_hbm.at[pl.ds(cid * N_ACC + r0, RPT)])

        zero_fire()
        load_idx(ujr_hbm, ujc_hbm, ujv_hbm)
        zero_drain()
        plsc.subcore_barrier()
        accumulate(bot_hbm)
        plsc.subcore_barrier()
        writeback(out_u_hbm)

        zero_fire()
        load_idx(ijr_hbm, ijc_hbm, ijv_hbm)
        zero_drain()
        plsc.subcore_barrier()
        accumulate(bot_hbm)
        plsc.subcore_barrier()
        writeback(out_t_hbm)

        zero_fire()
        load_idx(ujc_hbm, ujr_hbm, ujv_hbm)
        zero_drain()
        plsc.subcore_barrier()
        accumulate(usr_hbm)
        load_idx(ijc_hbm, ijr_hbm, ijv_hbm)
        accumulate(top_hbm)
        plsc.subcore_barrier()
        writeback(out_p_hbm)

    out_sds = jax.ShapeDtypeStruct((2 * N_ACC, DH), bf16)
    run = pl.kernel(
        body,
        out_type=(out_sds, out_sds, out_sds),
        mesh=mesh,
        compiler_params=pltpu.CompilerParams(use_tc_tiling_on_sc=False,
                                             needs_layout_passes=False),
        scratch_types=(
            pltpu.VMEM_SHARED((N_ACC, DH), bf16),
            pltpu.VMEM((CPT, C), jnp.int32),
            pltpu.VMEM((CPT, C), jnp.int32),
            pltpu.VMEM((CPT, C), f32),
            pltpu.VMEM((C, DH), bf16),
            pltpu.VMEM((C, DH), bf16),
            pltpu.VMEM((C, DH), bf16),
            pltpu.VMEM((C, DH), bf16),
            pltpu.VMEM((C, DH), bf16),
            pltpu.VMEM((C, DH), bf16),
            pltpu.SemaphoreType.DMA,
            pltpu.SemaphoreType.DMA,
            pltpu.SemaphoreType.DMA,
            pltpu.SemaphoreType.DMA,
            pltpu.SemaphoreType.DMA,
            pltpu.SemaphoreType.DMA,
            pltpu.SemaphoreType.DMA,
            pltpu.SemaphoreType.DMA,
            pltpu.SemaphoreType.DMA,
            pltpu.SemaphoreType.DMA,
            pltpu.SemaphoreType.DMA,
        ),
    )
    return run(bot, usr, top, ujr, ujc, ujv, ijr, ijc, ijv)


def kernel(adj_UJ_indices, adj_UJ_values, adj_IJ_indices, adj_IJ_values,
           top_embs, pos_bottoms_embs, all_users_embs):
    i32 = jnp.int32

    def pad_idx(x):
        return jnp.pad(x.astype(i32), (0, E_PAD - E)).reshape(E_PAD // C, C)

    def pad_val(x):
        return jnp.pad(x, (0, E_PAD - E)).reshape(E_PAD // C, C)

    ujr = pad_idx(adj_UJ_indices[0])
    ujc = pad_idx(adj_UJ_indices[1])
    ijr = pad_idx(adj_IJ_indices[0])
    ijc = pad_idx(adj_IJ_indices[1])
    ujv = pad_val(adj_UJ_values)
    ijv = pad_val(adj_IJ_values)

    def stack_halves(x):
        return jnp.concatenate([x[:, :DH], x[:, DH:]],
                               axis=0).astype(jnp.bfloat16)

    bot = stack_halves(pos_bottoms_embs)
    usr = stack_halves(all_users_embs)
    top = stack_halves(top_embs)

    out_u, out_t, out_p = _sc_lightgcn(bot, usr, top, ujr, ujc, ujv,
                                       ijr, ijc, ijv)

    def unstack(o):
        return jnp.concatenate([o[:N_ROWS], o[N_ACC:N_ACC + N_ROWS]],
                               axis=1).astype(jnp.float32)

    return (unstack(out_u), unstack(out_t), unstack(out_p))


# speedup vs baseline: 1.0479x; 1.0040x over previous
import jax
import jax.numpy as jnp
from jax import lax
from jax.experimental import pallas as pl
from jax.experimental.pallas import tpu as pltpu
from jax.experimental.pallas import tpu_sc as plsc

N_ROWS = 10000
N_ACC = 10240
D = 256
DH = 128
E = 160000
NT = 16
C = 128
CPT = 80
E_PAD = NT * CPT * C
RPT = N_ACC // NT


def _sc_lightgcn(bot, usr, top, ujr, ujc, ujv, ijr, ijc, ijv):
    mesh = plsc.VectorSubcoreMesh(core_axis_name="c", subcore_axis_name="s")
    f32 = jnp.float32
    bf16 = jnp.bfloat16

    def body(bot_hbm, usr_hbm, top_hbm,
             ujr_hbm, ujc_hbm, ujv_hbm, ijr_hbm, ijc_hbm, ijv_hbm,
             out_u_hbm, out_t_hbm, out_p_hbm,
             acc, rows_v, cols_v, vals_v, gb0, gb1, gb2, gb3, gb4, zbuf,
             sg0, sg1, sg2, sg3, sg4, ss0, ss1, ss2, ss3, ss4, sz):
        cid = lax.axis_index("c")
        tid = lax.axis_index("s")
        half_off = cid * N_ROWS
        gb = (gb0, gb1, gb2, gb3, gb4)
        sg = (sg0, sg1, sg2, sg3, sg4)
        ss = (ss0, ss1, ss2, ss3, ss4)

        z32 = jnp.zeros((32,), bf16)

        def zfill(r, carry):
            for c32 in range(DH // 32):
                zbuf[r, pl.ds(c32 * 32, 32)] = z32
            return carry
        lax.fori_loop(0, C, zfill, 0)

        def zero_fire():
            for k in range(RPT // C):
                pltpu.async_copy(zbuf, acc.at[pl.ds(tid * RPT + k * C, C)],
                                 sz)

        def zero_drain():
            for k in range(RPT // C):
                pltpu.make_async_copy(zbuf,
                                      acc.at[pl.ds(tid * RPT + k * C, C)],
                                      sz).wait()

        def load_idx(rows_hbm, cols_hbm, vals_hbm):
            base = tid * CPT
            pltpu.sync_copy(rows_hbm.at[pl.ds(base, CPT)], rows_v)
            pltpu.sync_copy(cols_hbm.at[pl.ds(base, CPT)], cols_v)
            pltpu.sync_copy(vals_hbm.at[pl.ds(base, CPT)], vals_v)

            off16 = jnp.full((16,), half_off, jnp.int32)

            def fix(i, carry):
                r = i // (C // 16)
                c = (i % (C // 16)) * 16
                cols_v[r, pl.ds(c, 16)] = cols_v[r, pl.ds(c, 16)] + off16
                return carry
            lax.fori_loop(0, CPT * (C // 16), fix, 0)

        def accumulate(table_hbm):

            def start_g(g, b):
                pltpu.async_copy(table_hbm.at[cols_v.at[g]], gb[b], sg[b])

            def wait_g(b):
                pltpu.make_async_copy(table_hbm.at[cols_v.at[0]], gb[b],
                                      sg[b]).wait()

            def start_s(g, b):
                pltpu.async_copy(gb[b], acc.at[rows_v.at[g]], ss[b], add=True)

            def wait_s(b):
                pltpu.make_async_copy(gb[b], acc.at[rows_v.at[0]],
                                      ss[b]).wait()

            dnums = lax.GatherDimensionNumbers(
                offset_dims=(), collapsed_slice_dims=(0,),
                start_index_map=(0,))

            def scale(g, b):
                buf = gb[b]

                def egroup(q, c2):
                    vv = vals_v[g, pl.ds(q * 16, 16)]

                    def lanes(lane, c3):
                        bidx = jnp.full((16,), lane, jnp.int32)
                        v16 = lax.gather(
                            vv, bidx[:, None], dnums, (1,),
                            mode=lax.GatherScatterMode.PROMISE_IN_BOUNDS)
                        v32 = plsc.pack(v16, v16,
                                        format=plsc.PackFormat.INTERLEAVED)
                        e = q * 16 + lane
                        for d32 in range(DH // 32):
                            sl = pl.ds(d32 * 32, 32)
                            buf[e, sl] = buf[e, sl] * v32
                        return c3
                    return lax.fori_loop(0, 16, lanes, c2)
                lax.fori_loop(0, C // 16, egroup, 0)

            start_g(0, 0)
            start_g(1, 1)
            start_g(2, 2)

            def rnd(r, carry):
                for j in range(5):
                    g = 5 * r + j
                    b = j
                    wait_g(b)

                    @pl.when(g >= 2)
                    def _():
                        wait_s((b + 3) % 5)

                    @pl.when(g + 3 < CPT)
                    def _():
                        start_g(g + 3, (b + 3) % 5)

                    scale(g, b)
                    start_s(g, b)
                return carry
            lax.fori_loop(0, CPT // 5, rnd, 0)
            wait_s((CPT - 2) % 5)
            wait_s((CPT - 1) % 5)

        def writeback(out_hbm):
            r0 = tid * RPT
            pltpu.sync_copy(acc.at[pl.ds(r0, RPT)],
                            out_hbm.at[pl.ds(cid * N_ACC + r0, RPT)])

        zero_fire()
        load_idx(ujr_hbm, ujc_hbm, ujv_hbm)
        zero_drain()
        plsc.subcore_barrier()
        accumulate(bot_hbm)
        plsc.subcore_barrier()
        writeback(out_u_hbm)

        zero_fire()
        load_idx(ijr_hbm, ijc_hbm, ijv_hbm)
        zero_drain()
        plsc.subcore_barrier()
        accumulate(bot_hbm)
        plsc.subcore_barrier()
        writeback(out_t_hbm)

        zero_fire()
        load_idx(ujc_hbm, ujr_hbm, ujv_hbm)
        zero_drain()
        plsc.subcore_barrier()
        accumulate(usr_hbm)
        load_idx(ijc_hbm, ijr_hbm, ijv_hbm)
        accumulate(top_hbm)
        plsc.subcore_barrier()
        writeback(out_p_hbm)

    out_sds = jax.ShapeDtypeStruct((2 * N_ACC, DH), bf16)
    run = pl.kernel(
        body,
        out_type=(out_sds, out_sds, out_sds),
        mesh=mesh,
        compiler_params=pltpu.CompilerParams(use_tc_tiling_on_sc=False,
                                             needs_layout_passes=False),
        scratch_types=(
            pltpu.VMEM_SHARED((N_ACC, DH), bf16),
            pltpu.VMEM((CPT, C), jnp.int32),
            pltpu.VMEM((CPT, C), jnp.int32),
            pltpu.VMEM((CPT, C), f32),
            pltpu.VMEM((C, DH), bf16),
            pltpu.VMEM((C, DH), bf16),
            pltpu.VMEM((C, DH), bf16),
            pltpu.VMEM((C, DH), bf16),
            pltpu.VMEM((C, DH), bf16),
            pltpu.VMEM((C, DH), bf16),
            pltpu.SemaphoreType.DMA,
            pltpu.SemaphoreType.DMA,
            pltpu.SemaphoreType.DMA,
            pltpu.SemaphoreType.DMA,
            pltpu.SemaphoreType.DMA,
            pltpu.SemaphoreType.DMA,
            pltpu.SemaphoreType.DMA,
            pltpu.SemaphoreType.DMA,
            pltpu.SemaphoreType.DMA,
            pltpu.SemaphoreType.DMA,
            pltpu.SemaphoreType.DMA,
        ),
    )
    return run(bot, usr, top, ujr, ujc, ujv, ijr, ijc, ijv)


def kernel(adj_UJ_indices, adj_UJ_values, adj_IJ_indices, adj_IJ_values,
           top_embs, pos_bottoms_embs, all_users_embs):
    i32 = jnp.int32

    def pad_idx(x):
        return jnp.pad(x.astype(i32), (0, E_PAD - E)).reshape(E_PAD // C, C)

    def pad_val(x):
        return jnp.pad(x, (0, E_PAD - E)).reshape(E_PAD // C, C)

    ujr = pad_idx(adj_UJ_indices[0])
    ujc = pad_idx(adj_UJ_indices[1])
    ijr = pad_idx(adj_IJ_indices[0])
    ijc = pad_idx(adj_IJ_indices[1])
    ujv = pad_val(adj_UJ_values)
    ijv = pad_val(adj_IJ_values)

    def stack_halves(x):
        return jnp.concatenate([x[:, :DH], x[:, DH:]],
                               axis=0).astype(jnp.bfloat16)

    bot = stack_halves(pos_bottoms_embs)
    usr = stack_halves(all_users_embs)
    top = stack_halves(top_embs)

    out_u, out_t, out_p = _sc_lightgcn(bot, usr, top, ujr, ujc, ujv,
                                       ijr, ijc, ijv)

    def unstack(o):
        return jnp.concatenate([o[:N_ROWS], o[N_ACC:N_ACC + N_ROWS]],
                               axis=1).astype(jnp.float32)

    return (unstack(out_u), unstack(out_t), unstack(out_p))
